# baseline (speedup 1.0000x reference)
import functools

import jax
import jax.numpy as jnp
from jax import lax
from jax.experimental import pallas as pl
from jax.experimental.pallas import tpu as pltpu
from jax.experimental.pallas import tpu_sc as plsc

D = 2048
S = 2048
B = 4
NC = 2
NS = 16
NW = NC * NS
CH = 8
NBUF = 4
LANES = 16
PER_W = S // NW
N_CHUNKS = PER_W // CH


def _sc_body(ids_hbm, wte_hbm, wpe_hbm, out_hbm, idx_v, rows0, rows1, rows2,
             rows3, wpe0, wpe1, gsem0, gsem1, gsem2, gsem3, ssem0, ssem1,
             ssem2, ssem3, wsem0, wsem1):
    wid = lax.axis_index("s") * NC + lax.axis_index("c")
    pos0 = wid * PER_W

    for b in range(B):
        pltpu.sync_copy(ids_hbm.at[pl.ds(b * S + pos0, PER_W)],
                        idx_v.at[pl.ds(b * PER_W, PER_W)])

    rows = (rows0, rows1, rows2, rows3)
    gsem = (gsem0, gsem1, gsem2, gsem3)
    ssem = (ssem0, ssem1, ssem2, ssem3)
    wpe = (wpe0, wpe1)
    wsem = (wsem0, wsem1)
    n_steps = N_CHUNKS * B

    def idx_slice(i):
        c, b = divmod(i, B)
        return idx_v.at[pl.ds(b * PER_W + c * CH, CH)]

    def out_slice(i):
        c, b = divmod(i, B)
        return out_hbm.at[pl.ds(b * S + pos0 + c * CH, CH)]

    def wpe_fetch(c):
        return pltpu.async_copy(wpe_hbm.at[pl.ds(pos0 + c * CH, CH)],
                                wpe[c % 2], wsem[c % 2])

    gat = [None] * NBUF
    sto = [None] * NBUF
    wfet = [None, None]
    wfet[0] = wpe_fetch(0)
    wfet[1] = wpe_fetch(1)
    gat[0] = pltpu.async_copy(wte_hbm.at[idx_slice(0)], rows[0], gsem[0])
    gat[1] = pltpu.async_copy(wte_hbm.at[idx_slice(1)], rows[1], gsem[1])

    for i in range(n_steps):
        p = i % NBUF
        if i + 2 < n_steps:
            q = (i + 2) % NBUF
            if sto[q] is not None:
                for d in sto[q]:
                    d.wait()
            gat[q] = pltpu.async_copy(wte_hbm.at[idx_slice(i + 2)], rows[q],
                                      gsem[q])
        c = i // B
        if i % B == 0:
            wfet[c % 2].wait()
        gat[p].wait()
        buf = rows[p]
        wbuf = wpe[c % 2]

        H = CH // 2
        halves = []
        for h in range(2):
            @plsc.parallel_loop(h * H, (h + 1) * H)
            def row(r):
                @plsc.parallel_loop(0, D, LANES, unroll=8)
                def col(j):
                    plsc.addupdate(buf.at[r, pl.ds(j, LANES)],
                                   wbuf[r, pl.ds(j, LANES)])

            halves.append(pltpu.async_copy(
                buf.at[pl.ds(h * H, H)], out_slice(i).at[pl.ds(h * H, H)],
                ssem[p]))
        sto[p] = halves
        if i % B == B - 1 and c + 2 < N_CHUNKS:
            wfet[c % 2] = wpe_fetch(c + 2)

    for p in range(NBUF):
        if sto[p] is not None:
            for d in sto[p]:
                d.wait()


def kernel(input_ids, wte, wpe):
    b, s = input_ids.shape
    ids_flat = input_ids.reshape(-1).astype(jnp.int32)
    n_rows = b * s

    mesh = plsc.VectorSubcoreMesh(core_axis_name="c", subcore_axis_name="s")
    out = pl.kernel(
        _sc_body,
        out_type=jax.ShapeDtypeStruct((n_rows, D), jnp.float32),
        mesh=mesh,
        scratch_types=[
            pltpu.VMEM((B * PER_W,), jnp.int32),
            pltpu.VMEM((CH, D), jnp.float32),
            pltpu.VMEM((CH, D), jnp.float32),
            pltpu.VMEM((CH, D), jnp.float32),
            pltpu.VMEM((CH, D), jnp.float32),
            pltpu.VMEM((CH, D), jnp.float32),
            pltpu.VMEM((CH, D), jnp.float32),
            pltpu.SemaphoreType.DMA,
            pltpu.SemaphoreType.DMA,
            pltpu.SemaphoreType.DMA,
            pltpu.SemaphoreType.DMA,
            pltpu.SemaphoreType.DMA,
            pltpu.SemaphoreType.DMA,
            pltpu.SemaphoreType.DMA,
            pltpu.SemaphoreType.DMA,
            pltpu.SemaphoreType.DMA,
            pltpu.SemaphoreType.DMA,
        ],
    )(ids_flat, wte, wpe)
    return out.reshape(b, s, D)


# speedup vs baseline: 1.0691x; 1.0691x over previous
import functools

import jax
import jax.numpy as jnp
from jax import lax
from jax.experimental import pallas as pl
from jax.experimental.pallas import tpu as pltpu
from jax.experimental.pallas import tpu_sc as plsc

D = 2048
S = 2048
B = 4
NC = 2
NS = 16
NW = NC * NS
CH = 8
NBUF = 5
LANES = 16
PER_W = S // NW
N_CHUNKS = PER_W // CH


def _sc_body(ids_hbm, wte_hbm, wpe_hbm, out_hbm, idx_v, rows0, rows1, rows2,
             rows3, rows4, wpe0, wpe1, gsem0, gsem1, gsem2, gsem3, gsem4,
             ssem0, ssem1, ssem2, ssem3, ssem4, wsem0, wsem1):
    wid = lax.axis_index("s") * NC + lax.axis_index("c")
    pos0 = wid * PER_W

    for b in range(B):
        pltpu.sync_copy(ids_hbm.at[pl.ds(b * S + pos0, PER_W)],
                        idx_v.at[pl.ds(b * PER_W, PER_W)])

    rows = (rows0, rows1, rows2, rows3, rows4)
    gsem = (gsem0, gsem1, gsem2, gsem3, gsem4)
    ssem = (ssem0, ssem1, ssem2, ssem3, ssem4)
    wpe = (wpe0, wpe1)
    wsem = (wsem0, wsem1)
    n_steps = N_CHUNKS * B

    def idx_slice(i):
        c, b = divmod(i, B)
        return idx_v.at[pl.ds(b * PER_W + c * CH, CH)]

    def out_slice(i):
        c, b = divmod(i, B)
        return out_hbm.at[pl.ds(b * S + pos0 + c * CH, CH)]

    def wpe_fetch(c):
        return pltpu.async_copy(wpe_hbm.at[pl.ds(pos0 + c * CH, CH)],
                                wpe[c % 2], wsem[c % 2])

    gat = [None] * NBUF
    sto = [None] * NBUF
    wfet = [None, None]
    wfet[0] = wpe_fetch(0)
    wfet[1] = wpe_fetch(1)
    for j in range(3):
        gat[j] = pltpu.async_copy(wte_hbm.at[idx_slice(j)], rows[j], gsem[j])

    for i in range(n_steps):
        p = i % NBUF
        if i + 3 < n_steps:
            q = (i + 3) % NBUF
            if sto[q] is not None:
                sto[q].wait()
            gat[q] = pltpu.async_copy(wte_hbm.at[idx_slice(i + 3)], rows[q],
                                      gsem[q])
        c = i // B
        if i % B == 0:
            wfet[c % 2].wait()
        gat[p].wait()
        buf = rows[p]
        wbuf = wpe[c % 2]

        @plsc.parallel_loop(0, CH)
        def row(r):
            @plsc.parallel_loop(0, D, LANES, unroll=8)
            def col(j):
                plsc.addupdate(buf.at[r, pl.ds(j, LANES)],
                               wbuf[r, pl.ds(j, LANES)])

        sto[p] = pltpu.async_copy(buf, out_slice(i), ssem[p])
        if i % B == B - 1 and c + 2 < N_CHUNKS:
            wfet[c % 2] = wpe_fetch(c + 2)

    for p in range(NBUF):
        if sto[p] is not None:
            sto[p].wait()


def kernel(input_ids, wte, wpe):
    b, s = input_ids.shape
    ids_flat = input_ids.reshape(-1).astype(jnp.int32)
    n_rows = b * s

    mesh = plsc.VectorSubcoreMesh(core_axis_name="c", subcore_axis_name="s")
    out = pl.kernel(
        _sc_body,
        out_type=jax.ShapeDtypeStruct((n_rows, D), jnp.float32),
        mesh=mesh,
        scratch_types=[
            pltpu.VMEM((B * PER_W,), jnp.int32),
            pltpu.VMEM((CH, D), jnp.float32),
            pltpu.VMEM((CH, D), jnp.float32),
            pltpu.VMEM((CH, D), jnp.float32),
            pltpu.VMEM((CH, D), jnp.float32),
            pltpu.VMEM((CH, D), jnp.float32),
            pltpu.VMEM((CH, D), jnp.float32),
            pltpu.VMEM((CH, D), jnp.float32),
            pltpu.SemaphoreType.DMA,
            pltpu.SemaphoreType.DMA,
            pltpu.SemaphoreType.DMA,
            pltpu.SemaphoreType.DMA,
            pltpu.SemaphoreType.DMA,
            pltpu.SemaphoreType.DMA,
            pltpu.SemaphoreType.DMA,
            pltpu.SemaphoreType.DMA,
            pltpu.SemaphoreType.DMA,
            pltpu.SemaphoreType.DMA,
            pltpu.SemaphoreType.DMA,
            pltpu.SemaphoreType.DMA,
        ],
    )(ids_flat, wte, wpe)
    return out.reshape(b, s, D)
